# Initial kernel scaffold; baseline (speedup 1.0000x reference)
import jax
import jax.numpy as jnp
from jax.experimental import pallas as pl


def kernel(demo, times, values, measurements):
    raise NotImplementedError("write your pallas kernel here")


# speedup vs baseline: 2.2099x; 2.2099x over previous
import functools

import jax
import jax.numpy as jnp
from jax import lax
from jax.experimental import pallas as pl
from jax.experimental.pallas import tpu as pltpu
from jax.experimental.pallas import tpu_sc as plsc

NC, NS, L = 2, 16, 16

_B, _T, _V, _D = 8, 2048, 16, 8
_N = _B * _T
_DEPTH = _D + _V
_W = _DEPTH + 2
_CHUNK = 512
_NROWS = _V * _N + _D * _B


def _sc_body(times_hbm, vals_hbm, mask_hbm, demo_hbm, out_hbm,
             times_v, vals_v, mask_v, tc_v, vc_v, demo_v, row_v):
    wid = lax.axis_index("s") * NC + lax.axis_index("c")
    iota = lax.iota(jnp.int32, L)

    @pl.when(wid < _V)
    def _channel():
        x = wid
        pltpu.sync_copy(times_hbm, times_v)
        pltpu.sync_copy(vals_hbm.at[pl.ds(x * _N, _N)], vals_v)
        pltpu.sync_copy(mask_hbm.at[pl.ds(x * _N, _N)], mask_v)

        def comp(i, w):
            m = mask_v[pl.ds(i * L, L)]
            mb = m != 0
            inc = plsc.cumsum(m)
            idx = w + inc - m
            plsc.store_scatter(tc_v, [idx], times_v[pl.ds(i * L, L)], mask=mb)
            plsc.store_scatter(vc_v, [idx], vals_v[pl.ds(i * L, L)], mask=mb)
            return w + jnp.sum(m)

        n_x = lax.fori_loop(0, _N // L, comp, jnp.int32(0))

        zf32 = jnp.zeros((L,), jnp.float32)

        def zero(i, _):
            row_v[pl.ds(i * L, L)] = zf32
            return 0

        lax.fori_loop(0, _CHUNK * _W // L, zero, 0)

        col1 = 1 + x
        ones = jnp.ones((L,), jnp.float32)

        def chunk_body(cidx, _):
            def fill(i, _):
                r0 = cidx * _CHUNK + i * L
                rvec = r0 + iota
                valid = rvec < n_x
                t = tc_v[pl.ds(r0, L)]
                v = vc_v[pl.ds(r0, L)]
                pos = (i * L + iota) * _W
                plsc.store_scatter(row_v, [pos], jnp.where(valid, t, 0.0))
                plsc.store_scatter(row_v, [pos + col1],
                                   jnp.where(valid, ones, zf32))
                plsc.store_scatter(row_v, [pos + (_W - 1)],
                                   jnp.where(valid, v, 0.0))
                return 0

            lax.fori_loop(0, _CHUNK // L, fill, 0)
            off = (x * _N + cidx * _CHUNK) * _W
            pltpu.sync_copy(row_v, out_hbm.at[pl.ds(off, _CHUNK * _W)])
            return 0

        lax.fori_loop(0, _N // _CHUNK, chunk_body, 0)

    @pl.when(wid == _V)
    def _demo():
        pltpu.sync_copy(demo_hbm, demo_v)
        nd = _D * _B

        def zero(i, _):
            row_v[pl.ds(i * L, L)] = jnp.zeros((L,), jnp.float32)
            return 0

        lax.fori_loop(0, nd * _W // L, zero, 0)
        ones = jnp.ones((L,), jnp.float32)
        for i in range(nd // L):
            r = i * L + iota
            col = _V + lax.shift_right_logical(r, 3)
            vals = demo_v[pl.ds(i * L, L)]
            plsc.store_scatter(row_v, [r * _W + col], ones)
            plsc.store_scatter(row_v, [r * _W + (_W - 1)], vals)
        pltpu.sync_copy(row_v.at[pl.ds(0, nd * _W)],
                        out_hbm.at[pl.ds(_V * _N * _W, nd * _W)])


@functools.partial(
    pl.kernel,
    out_type=jax.ShapeDtypeStruct((_NROWS * _W,), jnp.float32),
    mesh=plsc.VectorSubcoreMesh(core_axis_name="c", subcore_axis_name="s"),
    compiler_params=pltpu.CompilerParams(needs_layout_passes=False),
    scratch_types=[
        pltpu.VMEM((_N,), jnp.float32),
        pltpu.VMEM((_N,), jnp.float32),
        pltpu.VMEM((_N,), jnp.int32),
        pltpu.VMEM((_N,), jnp.float32),
        pltpu.VMEM((_N,), jnp.float32),
        pltpu.VMEM((_D * _B,), jnp.float32),
        pltpu.VMEM((_CHUNK * _W,), jnp.float32),
    ],
)
def _sc_kernel(times_hbm, vals_hbm, mask_hbm, demo_hbm, out_hbm,
               times_v, vals_v, mask_v, tc_v, vc_v, demo_v, row_v):
    _sc_body(times_hbm, vals_hbm, mask_hbm, demo_hbm, out_hbm,
             times_v, vals_v, mask_v, tc_v, vc_v, demo_v, row_v)


def kernel(demo, times, values, measurements):
    timesf = times.reshape(-1)
    valsT = jnp.transpose(values, (2, 0, 1)).reshape(-1)
    maskT = jnp.transpose(measurements, (2, 0, 1)).reshape(-1)
    maski = maskT.astype(jnp.int32)
    demoT = demo.T.reshape(-1)
    flat = _sc_kernel(timesf, valsT, maski, demoT)
    return flat.reshape(_NROWS, _W)
